# baseline (speedup 1.0000x reference)
import functools

import jax
import jax.numpy as jnp
from jax import lax
from jax.experimental import pallas as pl
from jax.experimental.pallas import tpu as pltpu
from jax.experimental.pallas import tpu_sc as plsc

N = 10000
E = 320000
D = 128

NP = 10240
EP = 327680
CHUNK = 128
KCH = 80
ROWS_PER_TILE = NP // 16

N_PAD_NODES = NP - N

_MESH = plsc.VectorSubcoreMesh(core_axis_name="c", subcore_axis_name="s")


@functools.partial(
    pl.kernel,
    out_type=jax.ShapeDtypeStruct((2, NP, 1), jnp.float32),
    mesh=_MESH,
    scratch_types=[
        pltpu.VMEM_SHARED((NP, 1), jnp.float32),
        pltpu.VMEM((KCH, CHUNK), jnp.int32),
        pltpu.VMEM((CHUNK, 1), jnp.float32),
        pltpu.SemaphoreType.DMA,
    ],
)
def _sc_degree(dst_hbm, z1_hbm, ones_hbm, out_hbm, acc, dstb, ones, sem):
    cid = lax.axis_index("c")
    sid = lax.axis_index("s")
    wid = sid * 2 + cid
    r0 = sid * ROWS_PER_TILE

    pltpu.sync_copy(ones_hbm, ones)
    pltpu.sync_copy(dst_hbm.at[pl.ds(wid * KCH, KCH)], dstb)
    pltpu.sync_copy(z1_hbm.at[pl.ds(r0, ROWS_PER_TILE)],
                    acc.at[pl.ds(r0, ROWS_PER_TILE)])
    plsc.subcore_barrier()

    def fire(j, _):
        pltpu.async_copy(ones, acc.at[dstb.at[j]], sem, add=True)
        return 0

    def drain(j, _):
        pltpu.make_async_copy(ones, acc.at[dstb.at[j]], sem).wait()
        return 0

    lax.fori_loop(0, KCH, fire, 0)
    lax.fori_loop(0, KCH, drain, 0)
    plsc.subcore_barrier()
    pltpu.sync_copy(acc.at[pl.ds(r0, ROWS_PER_TILE)],
                    out_hbm.at[cid, pl.ds(r0, ROWS_PER_TILE)])


@functools.partial(
    pl.kernel,
    out_type=jax.ShapeDtypeStruct((2, 16, ROWS_PER_TILE, D), jnp.float32),
    mesh=_MESH,
    scratch_types=[
        pltpu.VMEM_SHARED((NP, D), jnp.float32),
        pltpu.VMEM((KCH // 2, CHUNK), jnp.int32),
        pltpu.VMEM((KCH // 2, CHUNK), jnp.int32),
        pltpu.VMEM((CHUNK, D), jnp.float32),
        pltpu.VMEM((CHUNK, D), jnp.float32),
        pltpu.SemaphoreType.DMA,
        pltpu.SemaphoreType.DMA,
        pltpu.SemaphoreType.DMA,
        pltpu.SemaphoreType.DMA,
    ],
)
def _sc_agg_wide(hs_hbm, src_hbm, dst_hbm, z2_hbm, out_hbm, acc, srcb, dstb,
                 rows0, rows1, g0, g1, s0, s1):
    cid = lax.axis_index("c")
    sid = lax.axis_index("s")
    wid = sid * 2 + cid
    r0 = sid * ROWS_PER_TILE
    rpp = KCH // 2

    pltpu.sync_copy(z2_hbm.at[pl.ds(r0, ROWS_PER_TILE)],
                    acc.at[pl.ds(r0, ROWS_PER_TILE)])
    plsc.subcore_barrier()

    def run(base0, n_phases):
        def phase(p, _):
            base = base0 + p * rpp
            pltpu.sync_copy(src_hbm.at[pl.ds(base, rpp)], srcb)
            pltpu.sync_copy(dst_hbm.at[pl.ds(base, rpp)], dstb)
            pltpu.async_copy(hs_hbm.at[srcb.at[0]], rows0, g0)

            def pair(j, _):
                c0 = 2 * j
                c1 = c0 + 1
                pltpu.make_async_copy(hs_hbm.at[srcb.at[c0]], rows0, g0).wait()

                @pl.when(j > 0)
                def _():
                    pltpu.make_async_copy(rows1, acc.at[dstb.at[c1]], s1).wait()

                pltpu.async_copy(hs_hbm.at[srcb.at[c1]], rows1, g1)
                pltpu.async_copy(rows0, acc.at[dstb.at[c0]], s0, add=True)
                pltpu.make_async_copy(hs_hbm.at[srcb.at[c1]], rows1, g1).wait()
                pltpu.make_async_copy(rows0, acc.at[dstb.at[c0]], s0).wait()

                @pl.when(j < rpp // 2 - 1)
                def _():
                    pltpu.async_copy(hs_hbm.at[srcb.at[c1 + 1]], rows0, g0)

                pltpu.async_copy(rows1, acc.at[dstb.at[c1]], s1, add=True)
                return 0

            lax.fori_loop(0, rpp // 2, pair, 0)
            pltpu.make_async_copy(rows1, acc.at[dstb.at[rpp - 1]], s1).wait()
            return 0

        lax.fori_loop(0, n_phases, phase, 0)

    run(wid * KCH, 2)
    plsc.subcore_barrier()
    pltpu.sync_copy(acc.at[pl.ds(r0, ROWS_PER_TILE)], out_hbm.at[cid, sid])


@functools.partial(
    pl.kernel,
    out_type=jax.ShapeDtypeStruct((2, NP), jnp.float32),
    mesh=_MESH,
    scratch_types=[
        pltpu.VMEM_SHARED((NP,), jnp.float32),
        pltpu.VMEM((KCH, CHUNK), jnp.int32),
        pltpu.VMEM((KCH, CHUNK), jnp.int32),
        pltpu.VMEM((KCH, CHUNK), jnp.float32),
        pltpu.SemaphoreType.DMA,
        pltpu.SemaphoreType.DMA,
    ],
)
def _sc_agg_scalar(ss_hbm, src_hbm, dst_hbm, z1_hbm, out_hbm, acc, srcb, dstb,
                   valsb, g, s):
    cid = lax.axis_index("c")
    sid = lax.axis_index("s")
    wid = sid * 2 + cid
    r0 = sid * ROWS_PER_TILE

    pltpu.sync_copy(src_hbm.at[pl.ds(wid * KCH, KCH)], srcb)
    pltpu.sync_copy(dst_hbm.at[pl.ds(wid * KCH, KCH)], dstb)
    pltpu.sync_copy(z1_hbm.at[pl.ds(r0, ROWS_PER_TILE)],
                    acc.at[pl.ds(r0, ROWS_PER_TILE)])
    plsc.subcore_barrier()

    def fire_gather(j, _):
        pltpu.async_copy(ss_hbm.at[srcb.at[j]], valsb.at[j], g)
        return 0

    def scatter(j, _):
        pltpu.make_async_copy(ss_hbm.at[srcb.at[j]], valsb.at[j], g).wait()
        pltpu.async_copy(valsb.at[j], acc.at[dstb.at[j]], s, add=True)
        return 0

    def drain(j, _):
        pltpu.make_async_copy(valsb.at[j], acc.at[dstb.at[j]], s).wait()
        return 0

    lax.fori_loop(0, KCH, fire_gather, 0)
    lax.fori_loop(0, KCH, scatter, 0)
    lax.fori_loop(0, KCH, drain, 0)
    plsc.subcore_barrier()
    pltpu.sync_copy(acc.at[pl.ds(r0, ROWS_PER_TILE)],
                    out_hbm.at[cid, pl.ds(r0, ROWS_PER_TILE)])


_BLK = 1024


def _tc_scale_matmul(x_ref, w_ref, deg_ref, hs_ref):
    dd = deg_ref[...]
    dis = lax.rsqrt(dd[0] + dd[1] + 1.0)
    h = jnp.dot(x_ref[...], w_ref[...], preferred_element_type=jnp.float32)
    hs_ref[...] = h * dis


def _tc_mid(a_ref, hs_ref, deg_ref, w2_ref, b1_ref, ss_ref):
    dd = deg_ref[...]
    dis = lax.rsqrt(dd[0] + dd[1] + 1.0)
    a = a_ref[...]
    agg = dis * (a[0] + a[1] + hs_ref[...])
    h2 = jnp.maximum(agg + b1_ref[...], 0.0)
    s = jnp.dot(h2, w2_ref[...], preferred_element_type=jnp.float32)
    ss_ref[...] = s * dis


def _tc_final(a_ref, ss_ref, deg_ref, b2_ref, o_ref):
    dd = deg_ref[...]
    dis = lax.rsqrt(dd[0] + dd[1] + 1.0)
    a = a_ref[...]
    z = dis * (a[0] + a[1] + ss_ref[...]) + b2_ref[...]
    o_ref[...] = jax.nn.sigmoid(z)


def _tc_final80(a_ref, ss_ref, deg_ref, b2_ref, o_ref):
    dd = deg_ref[...]
    dis = lax.rsqrt(dd[0] + dd[1] + 1.0)
    a = a_ref[...]
    z = dis * (a[0] + a[1] + ss_ref[...]) + b2_ref[...]
    o_ref[...] = jax.nn.sigmoid(z)


def kernel(x, edge_index, W1, b1, W2, b2):
    ei = edge_index.astype(jnp.int32)
    pad = N + (jnp.arange(EP - E, dtype=jnp.int32) % N_PAD_NODES)
    src2d = jnp.concatenate([ei[0], pad]).reshape(32 * KCH, CHUNK)
    dst2d = jnp.concatenate([ei[1], pad]).reshape(32 * KCH, CHUNK)
    x_p = jnp.pad(x, ((0, NP - N), (0, 0)))
    z1 = jnp.zeros((NP,), jnp.float32)
    z2 = jnp.zeros((NP, D), jnp.float32)
    ones = jnp.ones((CHUNK,), jnp.float32)

    deg = _sc_degree(dst2d, z1.reshape(NP, 1), ones.reshape(CHUNK, 1))

    hs = pl.pallas_call(
        _tc_scale_matmul,
        grid=(NP // _BLK,),
        in_specs=[
            pl.BlockSpec((_BLK, D), lambda i: (i, 0)),
            pl.BlockSpec((D, D), lambda i: (0, 0)),
            pl.BlockSpec((2, _BLK, 1), lambda i: (0, i, 0)),
        ],
        out_specs=pl.BlockSpec((_BLK, D), lambda i: (i, 0)),
        out_shape=jax.ShapeDtypeStruct((NP, D), jnp.float32),
    )(x_p, W1, deg)

    acc1 = _sc_agg_wide(hs, src2d, dst2d, z2).reshape(2, NP, D)

    ss = pl.pallas_call(
        _tc_mid,
        grid=(NP // _BLK,),
        in_specs=[
            pl.BlockSpec((2, _BLK, D), lambda i: (0, i, 0)),
            pl.BlockSpec((_BLK, D), lambda i: (i, 0)),
            pl.BlockSpec((2, _BLK, 1), lambda i: (0, i, 0)),
            pl.BlockSpec((D, 1), lambda i: (0, 0)),
            pl.BlockSpec((1, D), lambda i: (0, 0)),
        ],
        out_specs=pl.BlockSpec((_BLK, 1), lambda i: (i, 0)),
        out_shape=jax.ShapeDtypeStruct((NP, 1), jnp.float32),
    )(acc1, hs, deg, W2, b1.reshape(1, D))

    ss1 = ss.reshape(NP)
    acc2 = _sc_agg_scalar(ss1, src2d, dst2d, z1)

    out = pl.pallas_call(
        _tc_final80,
        in_specs=[
            pl.BlockSpec((2, 80, D), lambda: (0, 0, 0)),
            pl.BlockSpec((80, D), lambda: (0, 0)),
            pl.BlockSpec((2, 80, D), lambda: (0, 0, 0)),
            pl.BlockSpec((1, 1), lambda: (0, 0)),
        ],
        out_specs=pl.BlockSpec((80, D), lambda: (0, 0)),
        out_shape=jax.ShapeDtypeStruct((80, D), jnp.float32),
    )(acc2.reshape(2, 80, D), ss1.reshape(80, D), deg.reshape(2, 80, D),
      b2.reshape(1, 1))

    return out.reshape(NP, 1)[:N]


# speedup vs baseline: 1.7585x; 1.7585x over previous
import functools

import jax
import jax.numpy as jnp
from jax import lax
from jax.experimental import pallas as pl
from jax.experimental.pallas import tpu as pltpu
from jax.experimental.pallas import tpu_sc as plsc

N = 10000
E = 320000
D = 128

NP = 10240
EP = 327680
CHUNK = 128
KCH = 80
ROWS_PER_TILE = NP // 16

N_PAD_NODES = NP - N

_MESH = plsc.VectorSubcoreMesh(core_axis_name="c", subcore_axis_name="s")


@functools.partial(
    pl.kernel,
    out_type=jax.ShapeDtypeStruct((2, NP), jnp.float32),
    mesh=_MESH,
    scratch_types=[
        pltpu.VMEM_SHARED((NP,), jnp.float32),
        pltpu.VMEM((KCH, CHUNK), jnp.int32),
        pltpu.VMEM((CHUNK,), jnp.float32),
        pltpu.SemaphoreType.DMA,
    ],
)
def _sc_degree(dst_hbm, z1_hbm, ones_hbm, out_hbm, acc, dstb, ones, sem):
    cid = lax.axis_index("c")
    sid = lax.axis_index("s")
    wid = sid * 2 + cid
    r0 = sid * ROWS_PER_TILE

    pltpu.sync_copy(ones_hbm, ones)
    pltpu.sync_copy(dst_hbm.at[pl.ds(wid * KCH, KCH)], dstb)
    pltpu.sync_copy(z1_hbm.at[pl.ds(r0, ROWS_PER_TILE)],
                    acc.at[pl.ds(r0, ROWS_PER_TILE)])
    plsc.subcore_barrier()

    def fire(j, _):
        pltpu.async_copy(ones, acc.at[dstb.at[j]], sem, add=True)
        return 0

    def drain(j, _):
        pltpu.make_async_copy(ones, acc.at[dstb.at[j]], sem).wait()
        return 0

    lax.fori_loop(0, KCH, fire, 0)
    lax.fori_loop(0, KCH, drain, 0)
    plsc.subcore_barrier()
    pltpu.sync_copy(acc.at[pl.ds(r0, ROWS_PER_TILE)],
                    out_hbm.at[cid, pl.ds(r0, ROWS_PER_TILE)])


@functools.partial(
    pl.kernel,
    out_type=jax.ShapeDtypeStruct((2, 16, ROWS_PER_TILE, D), jnp.float32),
    mesh=_MESH,
    scratch_types=[
        pltpu.VMEM_SHARED((NP, D), jnp.float32),
        pltpu.VMEM((KCH // 2, CHUNK), jnp.int32),
        pltpu.VMEM((KCH // 2, CHUNK), jnp.int32),
        pltpu.VMEM((CHUNK, D), jnp.float32),
        pltpu.VMEM((CHUNK, D), jnp.float32),
        pltpu.SemaphoreType.DMA,
        pltpu.SemaphoreType.DMA,
        pltpu.SemaphoreType.DMA,
        pltpu.SemaphoreType.DMA,
    ],
)
def _sc_agg_wide(hs_hbm, src_hbm, dst_hbm, z2_hbm, out_hbm, acc, srcb, dstb,
                 rows0, rows1, g0, g1, s0, s1):
    cid = lax.axis_index("c")
    sid = lax.axis_index("s")
    wid = sid * 2 + cid
    r0 = sid * ROWS_PER_TILE
    rpp = KCH // 2

    pltpu.sync_copy(z2_hbm.at[pl.ds(r0, ROWS_PER_TILE)],
                    acc.at[pl.ds(r0, ROWS_PER_TILE)])
    plsc.subcore_barrier()

    def run(base0, n_phases):
        def phase(p, _):
            base = base0 + p * rpp
            pltpu.sync_copy(src_hbm.at[pl.ds(base, rpp)], srcb)
            pltpu.sync_copy(dst_hbm.at[pl.ds(base, rpp)], dstb)
            pltpu.async_copy(hs_hbm.at[srcb.at[0]], rows0, g0)

            def pair(j, _):
                c0 = 2 * j
                c1 = c0 + 1
                pltpu.make_async_copy(hs_hbm.at[srcb.at[c0]], rows0, g0).wait()

                @pl.when(j > 0)
                def _():
                    pltpu.make_async_copy(rows1, acc.at[dstb.at[c1]], s1).wait()

                pltpu.async_copy(hs_hbm.at[srcb.at[c1]], rows1, g1)
                pltpu.async_copy(rows0, acc.at[dstb.at[c0]], s0, add=True)
                pltpu.make_async_copy(hs_hbm.at[srcb.at[c1]], rows1, g1).wait()
                pltpu.make_async_copy(rows0, acc.at[dstb.at[c0]], s0).wait()

                @pl.when(j < rpp // 2 - 1)
                def _():
                    pltpu.async_copy(hs_hbm.at[srcb.at[c1 + 1]], rows0, g0)

                pltpu.async_copy(rows1, acc.at[dstb.at[c1]], s1, add=True)
                return 0

            lax.fori_loop(0, rpp // 2, pair, 0)
            pltpu.make_async_copy(rows1, acc.at[dstb.at[rpp - 1]], s1).wait()
            return 0

        lax.fori_loop(0, n_phases, phase, 0)

    run(wid * KCH, 2)
    plsc.subcore_barrier()
    pltpu.sync_copy(acc.at[pl.ds(r0, ROWS_PER_TILE)], out_hbm.at[cid, sid])


@functools.partial(
    pl.kernel,
    out_type=jax.ShapeDtypeStruct((2, NP), jnp.float32),
    mesh=_MESH,
    scratch_types=[
        pltpu.VMEM_SHARED((NP,), jnp.float32),
        pltpu.VMEM((KCH, CHUNK), jnp.int32),
        pltpu.VMEM((KCH, CHUNK), jnp.int32),
        pltpu.VMEM((KCH, CHUNK), jnp.float32),
        pltpu.SemaphoreType.DMA,
        pltpu.SemaphoreType.DMA,
    ],
)
def _sc_agg_scalar(ss_hbm, src_hbm, dst_hbm, z1_hbm, out_hbm, acc, srcb, dstb,
                   valsb, g, s):
    cid = lax.axis_index("c")
    sid = lax.axis_index("s")
    wid = sid * 2 + cid
    r0 = sid * ROWS_PER_TILE

    pltpu.sync_copy(src_hbm.at[pl.ds(wid * KCH, KCH)], srcb)
    pltpu.sync_copy(dst_hbm.at[pl.ds(wid * KCH, KCH)], dstb)
    pltpu.sync_copy(z1_hbm.at[pl.ds(r0, ROWS_PER_TILE)],
                    acc.at[pl.ds(r0, ROWS_PER_TILE)])
    plsc.subcore_barrier()

    def fire_gather(j, _):
        pltpu.async_copy(ss_hbm.at[srcb.at[j]], valsb.at[j], g)
        return 0

    def scatter(j, _):
        pltpu.make_async_copy(ss_hbm.at[srcb.at[j]], valsb.at[j], g).wait()
        pltpu.async_copy(valsb.at[j], acc.at[dstb.at[j]], s, add=True)
        return 0

    def drain(j, _):
        pltpu.make_async_copy(valsb.at[j], acc.at[dstb.at[j]], s).wait()
        return 0

    lax.fori_loop(0, KCH, fire_gather, 0)
    lax.fori_loop(0, KCH, scatter, 0)
    lax.fori_loop(0, KCH, drain, 0)
    plsc.subcore_barrier()
    pltpu.sync_copy(acc.at[pl.ds(r0, ROWS_PER_TILE)],
                    out_hbm.at[cid, pl.ds(r0, ROWS_PER_TILE)])


_BLK = 1024


def _tc_scale_matmul(x_ref, w_ref, deg_ref, hs_ref):
    dd = deg_ref[...]
    dis = lax.rsqrt(dd[0] + dd[1] + 1.0)
    h = jnp.dot(x_ref[...], w_ref[...], preferred_element_type=jnp.float32)
    hs_ref[...] = h * dis


def _tc_mid(a_ref, hs_ref, deg_ref, w2_ref, b1_ref, ss_ref):
    dd = deg_ref[...]
    dis = lax.rsqrt(dd[0] + dd[1] + 1.0)
    a = a_ref[...]
    agg = dis * (a[0] + a[1] + hs_ref[...])
    h2 = jnp.maximum(agg + b1_ref[...], 0.0)
    s = jnp.dot(h2, w2_ref[...], preferred_element_type=jnp.float32)
    ss_ref[...] = s * dis


def _tc_final(a_ref, ss_ref, deg_ref, b2_ref, o_ref):
    dd = deg_ref[...]
    dis = lax.rsqrt(dd[0] + dd[1] + 1.0)
    a = a_ref[...]
    z = dis * (a[0] + a[1] + ss_ref[...]) + b2_ref[...]
    o_ref[...] = jax.nn.sigmoid(z)


def _tc_final80(a_ref, ss_ref, deg_ref, b2_ref, o_ref):
    dd = deg_ref[...]
    dis = lax.rsqrt(dd[0] + dd[1] + 1.0)
    a = a_ref[...]
    z = dis * (a[0] + a[1] + ss_ref[...]) + b2_ref[...]
    o_ref[...] = jax.nn.sigmoid(z)


def kernel(x, edge_index, W1, b1, W2, b2):
    ei = edge_index.astype(jnp.int32)
    pad = N + (jnp.arange(EP - E, dtype=jnp.int32) % N_PAD_NODES)
    src2d = jnp.concatenate([ei[0], pad]).reshape(32 * KCH, CHUNK)
    dst2d = jnp.concatenate([ei[1], pad]).reshape(32 * KCH, CHUNK)
    x_p = jnp.pad(x, ((0, NP - N), (0, 0)))
    z1 = jnp.zeros((NP,), jnp.float32)
    z2 = jnp.zeros((NP, D), jnp.float32)
    ones = jnp.ones((CHUNK,), jnp.float32)

    deg2 = _sc_degree(dst2d, z1, ones)
    deg = deg2.reshape(2, NP, 1)

    hs = pl.pallas_call(
        _tc_scale_matmul,
        grid=(NP // _BLK,),
        in_specs=[
            pl.BlockSpec((_BLK, D), lambda i: (i, 0)),
            pl.BlockSpec((D, D), lambda i: (0, 0)),
            pl.BlockSpec((2, _BLK, 1), lambda i: (0, i, 0)),
        ],
        out_specs=pl.BlockSpec((_BLK, D), lambda i: (i, 0)),
        out_shape=jax.ShapeDtypeStruct((NP, D), jnp.float32),
    )(x_p, W1, deg)

    acc1 = _sc_agg_wide(hs, src2d, dst2d, z2).reshape(2, NP, D)

    ss = pl.pallas_call(
        _tc_mid,
        grid=(NP // _BLK,),
        in_specs=[
            pl.BlockSpec((2, _BLK, D), lambda i: (0, i, 0)),
            pl.BlockSpec((_BLK, D), lambda i: (i, 0)),
            pl.BlockSpec((2, _BLK, 1), lambda i: (0, i, 0)),
            pl.BlockSpec((D, 1), lambda i: (0, 0)),
            pl.BlockSpec((1, D), lambda i: (0, 0)),
        ],
        out_specs=pl.BlockSpec((_BLK, 1), lambda i: (i, 0)),
        out_shape=jax.ShapeDtypeStruct((NP, 1), jnp.float32),
    )(acc1, hs, deg, W2, b1.reshape(1, D))

    ss1 = ss.reshape(NP)
    acc2 = _sc_agg_scalar(ss1, src2d, dst2d, z1)

    out = pl.pallas_call(
        _tc_final80,
        in_specs=[
            pl.BlockSpec((2, 80, D), lambda: (0, 0, 0)),
            pl.BlockSpec((80, D), lambda: (0, 0)),
            pl.BlockSpec((2, 80, D), lambda: (0, 0, 0)),
            pl.BlockSpec((1, 1), lambda: (0, 0)),
        ],
        out_specs=pl.BlockSpec((80, D), lambda: (0, 0)),
        out_shape=jax.ShapeDtypeStruct((80, D), jnp.float32),
    )(acc2.reshape(2, 80, D), ss1.reshape(80, D), deg2.reshape(2, 80, D),
      b2.reshape(1, 1))

    return out.reshape(NP, 1)[:N]
